# Initial kernel scaffold; baseline (speedup 1.0000x reference)
import jax
import jax.numpy as jnp
from jax.experimental import pallas as pl


def kernel(x, pos, edge_index_intra, x_bond, edge_index_inter, params):
    raise NotImplementedError("write your pallas kernel here")


# speedup vs baseline: 2.3253x; 2.3253x over previous
import functools

import jax
import jax.numpy as jnp
from jax import lax
from jax.experimental import pallas as pl
from jax.experimental.pallas import tpu as pltpu
from jax.experimental.pallas import tpu_sc as plsc

N = 10000
NPAD = 10240
E = 320000
H = 128
IN_DIM = 35
NRBF = 64
NC, NS = 2, 16
NW = NC * NS
TE = E // NW
K = 80
NCHUNK = TE // K
ZR = 16
RB = 512
GN = NPAD // RB
GE = E // RB
P4 = 16

_f32 = jnp.float32

_mesh = plsc.VectorSubcoreMesh(core_axis_name="c", subcore_axis_name="s",
                               num_cores=NC, num_subcores=NS)


def _sc_prep_body(px_hbm, py_hbm, pz_hbm, srcc, dstc, srcn, dstn,
                  d2c_o, d2n_o, degc_o, degn_o,
                  sidx, didx, px_v, py_v, pz_v, d2_v, ones_v, zbuf,
                  degacc, sem):
    cid = lax.axis_index("c")
    sid = lax.axis_index("s")
    wid = cid * NS + sid
    zeros16 = jnp.zeros((16,), _f32)
    ones16 = jnp.ones((16,), _f32)

    pltpu.sync_copy(px_hbm, px_v)
    pltpu.sync_copy(py_hbm, py_v)
    pltpu.sync_copy(pz_hbm, pz_v)

    @pl.loop(0, ZR)
    def _init(r):
        for c in range(H // 16):
            zbuf[r, pl.ds(c * 16, 16)] = zeros16

    @pl.loop(0, K)
    def _init2(r):
        for c in range(H // 16):
            ones_v[r, pl.ds(c * 16, 16)] = ones16

    rpt = NPAD // NS

    for src, dst, d2_o, deg_o in ((srcc, dstc, d2c_o, degc_o),
                                  (srcn, dstn, d2n_o, degn_o)):
        @pl.loop(0, rpt // ZR)
        def _zacc(j):
            pltpu.sync_copy(zbuf, degacc.at[pl.ds(sid * rpt + j * ZR, ZR)])

        plsc.subcore_barrier()

        @pl.loop(0, NCHUNK)
        def _chunk(i):
            base = wid * TE + i * K
            pltpu.sync_copy(src.at[pl.ds(base, K)], sidx)
            pltpu.sync_copy(dst.at[pl.ds(base, K)], didx)
            for j in range(K // 16):
                iv_s = sidx[pl.ds(j * 16, 16)]
                iv_d = didx[pl.ds(j * 16, 16)]
                dx = (plsc.load_gather(px_v, [iv_s])
                      - plsc.load_gather(px_v, [iv_d]))
                dy = (plsc.load_gather(py_v, [iv_s])
                      - plsc.load_gather(py_v, [iv_d]))
                dz = (plsc.load_gather(pz_v, [iv_s])
                      - plsc.load_gather(pz_v, [iv_d]))
                d2_v[pl.ds(j * 16, 16)] = dx * dx + dy * dy + dz * dz
            pltpu.sync_copy(d2_v, d2_o.at[pl.ds(base, K)])
            pltpu.sync_copy(ones_v, degacc.at[didx], add=True)

        plsc.subcore_barrier()

        @pl.loop(0, rpt // ZR)
        def _dump(j):
            r0 = sid * rpt + j * ZR
            pltpu.sync_copy(degacc.at[pl.ds(r0, ZR)],
                            deg_o.at[pl.ds(cid * NPAD + r0, ZR)])

        plsc.subcore_barrier()


_sc_prep = functools.partial(
    pl.kernel,
    out_type=(jax.ShapeDtypeStruct((E,), _f32),
              jax.ShapeDtypeStruct((E,), _f32),
              jax.ShapeDtypeStruct((NC * NPAD, H), _f32),
              jax.ShapeDtypeStruct((NC * NPAD, H), _f32)),
    mesh=_mesh,
    scratch_types=[pltpu.VMEM((K,), jnp.int32),
                   pltpu.VMEM((K,), jnp.int32),
                   pltpu.VMEM((NPAD,), _f32),
                   pltpu.VMEM((NPAD,), _f32),
                   pltpu.VMEM((NPAD,), _f32),
                   pltpu.VMEM((K,), _f32),
                   pltpu.VMEM((K, H), _f32),
                   pltpu.VMEM((ZR, H), _f32),
                   pltpu.VMEM_SHARED((NPAD, H), _f32),
                   pltpu.SemaphoreType.DMA],
    compiler_params=pltpu.CompilerParams(needs_layout_passes=False),
)(_sc_prep_body)


def _sc_edge_body(src, dst, ai_t, aj_t, e_t, out_o,
                  sidx, didx, ai_v, aj_v, e_v, zbuf, acc, sem):
    cid = lax.axis_index("c")
    sid = lax.axis_index("s")
    wid = cid * NS + sid
    zeros16 = jnp.zeros((16,), _f32)

    @pl.loop(0, ZR)
    def _z(r):
        for c in range(H // 16):
            zbuf[r, pl.ds(c * 16, 16)] = zeros16

    rpt = NPAD // NS

    @pl.loop(0, rpt // ZR)
    def _zacc(j):
        pltpu.sync_copy(zbuf, acc.at[pl.ds(sid * rpt + j * ZR, ZR)])

    plsc.subcore_barrier()

    @pl.loop(0, NCHUNK)
    def _chunk(i):
        base = wid * TE + i * K
        pltpu.sync_copy(src.at[pl.ds(base, K)], sidx)
        pltpu.sync_copy(dst.at[pl.ds(base, K)], didx)
        c1 = pltpu.async_copy(ai_t.at[didx], ai_v, sem)
        c2 = pltpu.async_copy(aj_t.at[sidx], aj_v, sem)
        c3 = pltpu.async_copy(e_t.at[pl.ds(base, K)], e_v, sem)
        c1.wait()
        c2.wait()
        c3.wait()

        @pl.loop(0, K)
        def _row(r):
            for c in range(H // 16):
                sl = pl.ds(c * 16, 16)
                s = ai_v[r, sl] + aj_v[r, sl] + e_v[r, sl]
                ai_v[r, sl] = jnp.maximum(s, 0.0)

        pltpu.sync_copy(ai_v, acc.at[didx], add=True)

    plsc.subcore_barrier()

    @pl.loop(0, rpt // ZR)
    def _dump(j):
        r0 = sid * rpt + j * ZR
        pltpu.sync_copy(acc.at[pl.ds(r0, ZR)],
                        out_o.at[pl.ds(cid * NPAD + r0, ZR)])


_sc_edge = functools.partial(
    pl.kernel,
    out_type=jax.ShapeDtypeStruct((NC * NPAD, H), _f32),
    mesh=_mesh,
    scratch_types=[pltpu.VMEM((K,), jnp.int32),
                   pltpu.VMEM((K,), jnp.int32),
                   pltpu.VMEM((K, H), _f32),
                   pltpu.VMEM((K, H), _f32),
                   pltpu.VMEM((K, H), _f32),
                   pltpu.VMEM((ZR, H), _f32),
                   pltpu.VMEM_SHARED((NPAD, H), _f32),
                   pltpu.SemaphoreType.DMA],
    compiler_params=pltpu.CompilerParams(needs_layout_passes=False),
)(_sc_edge_body)


def _h0_body(x_ref, w_ref, b_ref, o_ref):
    o_ref[...] = (jnp.dot(x_ref[...], w_ref[...],
                          preferred_element_type=_f32, precision=jax.lax.Precision.HIGHEST) + b_ref[...])


def _a_body(h_ref, w_ref, o0, o1, o2, o3):
    hb = h_ref[...]
    for idx, o in enumerate((o0, o1, o2, o3)):
        o[...] = jnp.dot(hb, w_ref[idx], preferred_element_type=_f32, precision=jax.lax.Precision.HIGHEST)


def _e_body(d2c_r, xb, d2n_r,
            cc, c2, c6, wc, w2, w6,
            rbfWc, bondW, bconstc, rbfW2, rbfW6, bconstn,
            ec0, ec1, ec2, en0, en1, en2):
    dc = jnp.sqrt(d2c_r[...])
    rb_c = jnp.exp(-((dc - cc[...]) ** 2) / wc[...])
    dn = jnp.sqrt(d2n_r[...])
    p2 = dn * dn
    p4 = p2 * p2
    u = 1.0 / p2
    v6 = 1.0 / (p4 * p2)
    rb2 = jnp.exp(-((u - c2[...]) ** 2) / w2[...])
    rb6 = jnp.exp(-((v6 - c6[...]) ** 2) / w6[...])
    xbb = xb[...]
    for l, eo in enumerate((ec0, ec1, ec2)):
        eo[...] = (jnp.dot(xbb, bondW[l], preferred_element_type=_f32, precision=jax.lax.Precision.HIGHEST)
                   + jnp.dot(rb_c, rbfWc[l], preferred_element_type=_f32, precision=jax.lax.Precision.HIGHEST)
                   + bconstc[l])
    for l, eo in enumerate((en0, en1, en2)):
        eo[...] = (jnp.dot(rb2, rbfW2[l], preferred_element_type=_f32, precision=jax.lax.Precision.HIGHEST)
                   + jnp.dot(rb6, rbfW6[l], preferred_element_type=_f32, precision=jax.lax.Precision.HIGHEST)
                   + bconstn[l])


def _upd_body(h_ref, s0c, s1c, s0n, s1n, dgc, dgn,
              w2c, b2c, w2n, b2n, wuc, buc, wun, bun,
              ho, cs):
    i = pl.program_id(0)
    hb = h_ref[...]
    Sc = s0c[...] + s1c[...]
    mc = jnp.dot(Sc, w2c[...], preferred_element_type=_f32, precision=jax.lax.Precision.HIGHEST) + dgc[...] * b2c[...]
    Sn = s0n[...] + s1n[...]
    mn = jnp.dot(Sn, w2n[...], preferred_element_type=_f32, precision=jax.lax.Precision.HIGHEST) + dgn[...] * b2n[...]
    hc = jnp.maximum(jnp.dot(hb + mc, wuc[...],
                             preferred_element_type=_f32, precision=jax.lax.Precision.HIGHEST) + buc[...], 0.0)
    hn = jnp.maximum(jnp.dot(hb + mn, wun[...],
                             preferred_element_type=_f32, precision=jax.lax.Precision.HIGHEST) + bun[...], 0.0)
    hnew = hc + hn
    ho[...] = hnew
    rows = i * RB + lax.broadcasted_iota(jnp.int32, (RB, 1), 0)
    cs[...] = jnp.sum(jnp.where(rows < N, hnew, 0.0),
                      axis=0, keepdims=True)[None]


def _head_body(parts, qkvW, qkvb, outW, outb, mW1, mb1, mw2, mb2, o_ref):
    r = jnp.sum(parts[...], axis=0)
    qkv = jnp.dot(r, qkvW[...], preferred_element_type=_f32, precision=jax.lax.Precision.HIGHEST) + qkvb[...]
    v = jnp.concatenate([qkv[:, 96 * t + 64:96 * t + 96] for t in range(4)],
                        axis=1)
    fin = jnp.dot(v, outW[...], preferred_element_type=_f32, precision=jax.lax.Precision.HIGHEST) + outb[...]
    o = jnp.maximum(jnp.dot(fin, mW1[...],
                            preferred_element_type=_f32, precision=jax.lax.Precision.HIGHEST) + mb1[...], 0.0)
    o_ref[...] = jnp.sum(o * mw2[...], axis=1, keepdims=True) + mb2[...]


def _row_spec(i):
    return (i, 0)


def _fix_spec(i):
    return (0, 0)


_h0_call = pl.pallas_call(
    _h0_body,
    grid=(GN,),
    in_specs=[pl.BlockSpec((RB, IN_DIM), _row_spec),
              pl.BlockSpec((IN_DIM, H), _fix_spec),
              pl.BlockSpec((1, H), _fix_spec)],
    out_specs=pl.BlockSpec((RB, H), _row_spec),
    out_shape=jax.ShapeDtypeStruct((NPAD, H), _f32),
)

_a_call = pl.pallas_call(
    _a_body,
    grid=(GN,),
    in_specs=[pl.BlockSpec((RB, H), _row_spec),
              pl.BlockSpec((4, H, H), lambda i: (0, 0, 0))],
    out_specs=[pl.BlockSpec((RB, H), _row_spec)] * 4,
    out_shape=[jax.ShapeDtypeStruct((NPAD, H), _f32)] * 4,
)

_e_call = pl.pallas_call(
    _e_body,
    grid=(GE,),
    in_specs=[pl.BlockSpec((RB, 1), _row_spec),
              pl.BlockSpec((RB, 10), _row_spec),
              pl.BlockSpec((RB, 1), _row_spec),
              pl.BlockSpec((1, NRBF), _fix_spec),
              pl.BlockSpec((1, NRBF), _fix_spec),
              pl.BlockSpec((1, NRBF), _fix_spec),
              pl.BlockSpec((1, 1), _fix_spec),
              pl.BlockSpec((1, 1), _fix_spec),
              pl.BlockSpec((1, 1), _fix_spec),
              pl.BlockSpec((3, NRBF, H), lambda i: (0, 0, 0)),
              pl.BlockSpec((3, 10, H), lambda i: (0, 0, 0)),
              pl.BlockSpec((3, 1, H), lambda i: (0, 0, 0)),
              pl.BlockSpec((3, NRBF, H), lambda i: (0, 0, 0)),
              pl.BlockSpec((3, NRBF, H), lambda i: (0, 0, 0)),
              pl.BlockSpec((3, 1, H), lambda i: (0, 0, 0))],
    out_specs=[pl.BlockSpec((RB, H), _row_spec)] * 6,
    out_shape=[jax.ShapeDtypeStruct((E, H), _f32)] * 6,
)

_upd_call = pl.pallas_call(
    _upd_body,
    grid=(GN,),
    in_specs=[pl.BlockSpec((RB, H), _row_spec),
              pl.BlockSpec((RB, H), _row_spec),
              pl.BlockSpec((RB, H), lambda i: (GN + i, 0)),
              pl.BlockSpec((RB, H), _row_spec),
              pl.BlockSpec((RB, H), lambda i: (GN + i, 0)),
              pl.BlockSpec((RB, 1), _row_spec),
              pl.BlockSpec((RB, 1), _row_spec),
              pl.BlockSpec((H, H), _fix_spec),
              pl.BlockSpec((1, H), _fix_spec),
              pl.BlockSpec((H, H), _fix_spec),
              pl.BlockSpec((1, H), _fix_spec),
              pl.BlockSpec((H, H), _fix_spec),
              pl.BlockSpec((1, H), _fix_spec),
              pl.BlockSpec((H, H), _fix_spec),
              pl.BlockSpec((1, H), _fix_spec)],
    out_specs=[pl.BlockSpec((RB, H), _row_spec),
               pl.BlockSpec((1, 1, H), lambda i: (i, 0, 0))],
    out_shape=[jax.ShapeDtypeStruct((NPAD, H), _f32),
               jax.ShapeDtypeStruct((GN, 1, H), _f32)],
)

_head_call = pl.pallas_call(
    _head_body,
    out_shape=jax.ShapeDtypeStruct((1, 1), _f32),
)


def kernel(x, pos, edge_index_intra, x_bond, edge_index_inter, params):
    src_c, dst_c = edge_index_intra[0], edge_index_intra[1]
    src_n, dst_n = edge_index_inter[0], edge_index_inter[1]

    posp = jnp.zeros((NPAD, 3), _f32).at[:N].set(pos)
    xpad = jnp.zeros((NPAD, IN_DIM), _f32).at[:N].set(x)

    d2c, d2n, degc_p, degn_p = _sc_prep(
        posp[:, 0], posp[:, 1], posp[:, 2], src_c, dst_c, src_n, dst_n)

    deg_c = (degc_p[:NPAD, :1] + degc_p[NPAD:, :1])
    deg_n = (degn_p[:NPAD, :1] + degn_p[NPAD:, :1])

    lp = params['layers']
    bondW = jnp.stack([l['bond_W'] @ l['cov_W1'][256:384] for l in lp])
    bconstc = jnp.stack([(l['bond_b'] @ l['cov_W1'][256:384]
                          + l['cov_b1'])[None, :] for l in lp])
    rbfWc = jnp.stack([l['cov_W1'][384:448] for l in lp])
    rbfW2 = jnp.stack([l['ncov_W1'][256:320] for l in lp])
    rbfW6 = jnp.stack([l['ncov_W1'][320:384] for l in lp])
    bconstn = jnp.stack([l['ncov_b1'][None, :] for l in lp])

    cc = jnp.linspace(jnp.float32(1.0), jnp.float32(6.0), NRBF,
                      dtype=_f32)
    c2 = jnp.linspace(jnp.float32(1.0), jnp.float32(6.0) ** -2, NRBF,
                      dtype=_f32)
    c6 = jnp.linspace(jnp.float32(1.0), jnp.float32(6.0) ** -6, NRBF,
                      dtype=_f32)
    wc = ((cc[1] - cc[0]) ** 2).reshape(1, 1)
    w2 = ((c2[1] - c2[0]) ** 2).reshape(1, 1)
    w6 = ((c6[1] - c6[0]) ** 2).reshape(1, 1)

    ec0, ec1, ec2, en0, en1, en2 = _e_call(
        d2c.reshape(E, 1), x_bond, d2n.reshape(E, 1),
        cc.reshape(1, NRBF), c2.reshape(1, NRBF), c6.reshape(1, NRBF),
        wc, w2, w6, rbfWc, bondW, bconstc, rbfW2, rbfW6, bconstn)
    e_cov = (ec0, ec1, ec2)
    e_ncov = (en0, en1, en2)

    h = _h0_call(xpad, params['atom_W'], params['atom_b'][None, :])

    cs = None
    for li, l in enumerate(lp):
        W4 = jnp.stack([l['cov_W1'][0:128], l['cov_W1'][128:256],
                        l['ncov_W1'][0:128], l['ncov_W1'][128:256]])
        a_ic, a_jc, a_in, a_jn = _a_call(h, W4)
        Sc = _sc_edge(src_c, dst_c, a_ic, a_jc, e_cov[li])
        Sn = _sc_edge(src_n, dst_n, a_in, a_jn, e_ncov[li])
        h, cs = _upd_call(h, Sc, Sc, Sn, Sn, deg_c, deg_n,
                          l['cov_W2'], l['cov_b2'][None, :],
                          l['ncov_W2'], l['ncov_b2'][None, :],
                          l['upc_W'], l['upc_b'][None, :],
                          l['upn_W'], l['upn_b'][None, :])

    out = _head_call(cs, params['qkv_W'], params['qkv_b'][None, :],
                     params['out_W'], params['out_b'][None, :],
                     params['mlp_W1'], params['mlp_b1'][None, :],
                     params['mlp_W2'].reshape(1, H),
                     params['mlp_b2'].reshape(1, 1))
    return out.reshape(-1)


# speedup vs baseline: 3.3506x; 1.4409x over previous
import functools

import jax
import jax.numpy as jnp
from jax import lax
from jax.experimental import pallas as pl
from jax.experimental.pallas import tpu as pltpu
from jax.experimental.pallas import tpu_sc as plsc

N = 10000
NPAD = 10240
E = 320000
H = 128
IN_DIM = 35
NRBF = 64
NC, NS = 2, 16
NW = NC * NS
TE = E // NW
K = 80
NCHUNK = TE // K
ZR = 16
RB = 512
GN = NPAD // RB
GE = E // RB
P4 = 16

_f32 = jnp.float32

_mesh = plsc.VectorSubcoreMesh(core_axis_name="c", subcore_axis_name="s",
                               num_cores=NC, num_subcores=NS)


def _sc_prep_body(px_hbm, py_hbm, pz_hbm, srcc, dstc, srcn, dstn,
                  d2c_o, d2n_o, degc_o, degn_o,
                  sidx, didx, px_v, py_v, pz_v, d2_v, ones_v, zbuf,
                  degacc, sem):
    cid = lax.axis_index("c")
    sid = lax.axis_index("s")
    wid = cid * NS + sid
    zeros16 = jnp.zeros((16,), _f32)
    ones16 = jnp.ones((16,), _f32)

    pltpu.sync_copy(px_hbm, px_v)
    pltpu.sync_copy(py_hbm, py_v)
    pltpu.sync_copy(pz_hbm, pz_v)

    @pl.loop(0, ZR)
    def _init(r):
        for c in range(H // 16):
            zbuf[r, pl.ds(c * 16, 16)] = zeros16

    @pl.loop(0, K)
    def _init2(r):
        for c in range(H // 16):
            ones_v[r, pl.ds(c * 16, 16)] = ones16

    rpt = NPAD // NS

    for src, dst, d2_o, deg_o in ((srcc, dstc, d2c_o, degc_o),
                                  (srcn, dstn, d2n_o, degn_o)):
        @pl.loop(0, rpt // ZR)
        def _zacc(j):
            pltpu.sync_copy(zbuf, degacc.at[pl.ds(sid * rpt + j * ZR, ZR)])

        plsc.subcore_barrier()

        @pl.loop(0, NCHUNK)
        def _chunk(i):
            base = wid * TE + i * K
            pltpu.sync_copy(src.at[pl.ds(base, K)], sidx)
            pltpu.sync_copy(dst.at[pl.ds(base, K)], didx)
            for j in range(K // 16):
                iv_s = sidx[pl.ds(j * 16, 16)]
                iv_d = didx[pl.ds(j * 16, 16)]
                dx = (plsc.load_gather(px_v, [iv_s])
                      - plsc.load_gather(px_v, [iv_d]))
                dy = (plsc.load_gather(py_v, [iv_s])
                      - plsc.load_gather(py_v, [iv_d]))
                dz = (plsc.load_gather(pz_v, [iv_s])
                      - plsc.load_gather(pz_v, [iv_d]))
                d2_v[pl.ds(j * 16, 16)] = dx * dx + dy * dy + dz * dz
            pltpu.sync_copy(d2_v, d2_o.at[pl.ds(base, K)])
            pltpu.sync_copy(ones_v, degacc.at[didx], add=True)

        plsc.subcore_barrier()

        @pl.loop(0, rpt // ZR)
        def _dump(j):
            r0 = sid * rpt + j * ZR
            pltpu.sync_copy(degacc.at[pl.ds(r0, ZR)],
                            deg_o.at[pl.ds(cid * NPAD + r0, ZR)])

        plsc.subcore_barrier()


_sc_prep = functools.partial(
    pl.kernel,
    out_type=(jax.ShapeDtypeStruct((E,), _f32),
              jax.ShapeDtypeStruct((E,), _f32),
              jax.ShapeDtypeStruct((NC * NPAD, H), _f32),
              jax.ShapeDtypeStruct((NC * NPAD, H), _f32)),
    mesh=_mesh,
    scratch_types=[pltpu.VMEM((K,), jnp.int32),
                   pltpu.VMEM((K,), jnp.int32),
                   pltpu.VMEM((NPAD,), _f32),
                   pltpu.VMEM((NPAD,), _f32),
                   pltpu.VMEM((NPAD,), _f32),
                   pltpu.VMEM((K,), _f32),
                   pltpu.VMEM((K, H), _f32),
                   pltpu.VMEM((ZR, H), _f32),
                   pltpu.VMEM_SHARED((NPAD, H), _f32),
                   pltpu.SemaphoreType.DMA],
    compiler_params=pltpu.CompilerParams(needs_layout_passes=False),
)(_sc_prep_body)


K2 = 40
NCHUNK2 = TE // K2


def _sc_edge_body(srcc, dstc, aic_t, ajc_t, ec_t,
                  srcn, dstn, ain_t, ajn_t, en_t, out_o,
                  sA, dA, sB, dB, ai0, aj0, e0, ai1, aj1, e1,
                  acc, semIA, semIB, semA, semB):
    cid = lax.axis_index("c")
    sid = lax.axis_index("s")
    wid = cid * NS + sid
    zeros16 = jnp.zeros((16,), _f32)
    rpt = NPAD // NS
    tb = wid * TE

    for bi, (src, dst, ai_t, aj_t, e_t, ob) in enumerate(
            ((srcc, dstc, aic_t, ajc_t, ec_t, 0),
             (srcn, dstn, ain_t, ajn_t, en_t, NC * NPAD))):
        @pl.loop(0, K2)
        def _z(r):
            for c in range(H // 16):
                e0[r, pl.ds(c * 16, 16)] = zeros16

        @pl.loop(0, rpt // K2)
        def _zacc(j):
            pltpu.sync_copy(e0, acc.at[pl.ds(sid * rpt + j * K2, K2)])

        plsc.subcore_barrier()

        def issue_idx(c, sb, db, sem):
            cc = jnp.minimum(c, NCHUNK2 - 1)
            pltpu.async_copy(src.at[pl.ds(tb + cc * K2, K2)], sb, sem)
            pltpu.async_copy(dst.at[pl.ds(tb + cc * K2, K2)], db, sem)

        def drain_idx(sb, db, sem):
            pltpu.make_async_copy(src.at[pl.ds(0, K2)], sb, sem).wait()
            pltpu.make_async_copy(dst.at[pl.ds(0, K2)], db, sem).wait()

        def issue_g(c, sb, db, ai_b, aj_b, e_b, sem):
            cc = jnp.minimum(c, NCHUNK2 - 1)
            pltpu.async_copy(ai_t.at[db], ai_b, sem)
            pltpu.async_copy(aj_t.at[sb], aj_b, sem)
            pltpu.async_copy(e_t.at[pl.ds(tb + cc * K2, K2)], e_b, sem)

        def drain_g(ai_b, aj_b, e_b, sem):
            pltpu.make_async_copy(ai_t.at[pl.ds(0, K2)], ai_b, sem).wait()
            pltpu.make_async_copy(aj_t.at[pl.ds(0, K2)], aj_b, sem).wait()
            pltpu.make_async_copy(e_t.at[pl.ds(0, K2)], e_b, sem).wait()

        def compute_scatter(db, ai_b, aj_b, e_b):
            @pl.loop(0, K2)
            def _row(r):
                for cc in range(H // 16):
                    sl = pl.ds(cc * 16, 16)
                    s = ai_b[r, sl] + aj_b[r, sl] + e_b[r, sl]
                    ai_b[r, sl] = jnp.maximum(s, 0.0)

            pltpu.sync_copy(ai_b, acc.at[db], add=True)

        issue_idx(0, sA, dA, semIA)
        issue_idx(1, sB, dB, semIB)
        drain_idx(sA, dA, semIA)
        issue_g(0, sA, dA, ai0, aj0, e0, semA)

        @pl.loop(0, NCHUNK2 // 2)
        def _pair(k):
            c0 = 2 * k
            drain_g(ai0, aj0, e0, semA)
            drain_idx(sB, dB, semIB)
            issue_g(c0 + 1, sB, dB, ai1, aj1, e1, semB)
            compute_scatter(dA, ai0, aj0, e0)
            issue_idx(c0 + 2, sA, dA, semIA)
            drain_g(ai1, aj1, e1, semB)
            drain_idx(sA, dA, semIA)
            issue_g(c0 + 2, sA, dA, ai0, aj0, e0, semA)
            compute_scatter(dB, ai1, aj1, e1)
            issue_idx(c0 + 3, sB, dB, semIB)

        drain_g(ai0, aj0, e0, semA)
        drain_idx(sB, dB, semIB)

        plsc.subcore_barrier()

        pltpu.sync_copy(acc.at[pl.ds(sid * rpt, rpt)],
                        out_o.at[pl.ds(ob + cid * NPAD + sid * rpt, rpt)])

        plsc.subcore_barrier()


_sc_edge = functools.partial(
    pl.kernel,
    out_type=jax.ShapeDtypeStruct((2 * NC * NPAD, H), _f32),
    mesh=_mesh,
    scratch_types=[pltpu.VMEM((K2,), jnp.int32),
                   pltpu.VMEM((K2,), jnp.int32),
                   pltpu.VMEM((K2,), jnp.int32),
                   pltpu.VMEM((K2,), jnp.int32),
                   pltpu.VMEM((K2, H), _f32),
                   pltpu.VMEM((K2, H), _f32),
                   pltpu.VMEM((K2, H), _f32),
                   pltpu.VMEM((K2, H), _f32),
                   pltpu.VMEM((K2, H), _f32),
                   pltpu.VMEM((K2, H), _f32),
                   pltpu.VMEM_SHARED((NPAD, H), _f32),
                   pltpu.SemaphoreType.DMA,
                   pltpu.SemaphoreType.DMA,
                   pltpu.SemaphoreType.DMA,
                   pltpu.SemaphoreType.DMA],
    compiler_params=pltpu.CompilerParams(needs_layout_passes=False),
)(_sc_edge_body)


def _h0_body(x_ref, w_ref, b_ref, o_ref):
    o_ref[...] = (jnp.dot(x_ref[...], w_ref[...],
                          preferred_element_type=_f32, precision=jax.lax.Precision.HIGHEST) + b_ref[...])


def _a_body(h_ref, w_ref, o0, o1, o2, o3):
    hb = h_ref[...]
    for idx, o in enumerate((o0, o1, o2, o3)):
        o[...] = jnp.dot(hb, w_ref[idx], preferred_element_type=_f32, precision=jax.lax.Precision.HIGHEST)


def _e_body(d2c_r, xb, d2n_r,
            cc, c2, c6, wc, w2, w6,
            rbfWc, bondW, bconstc, rbfW2, rbfW6, bconstn,
            ec, en):
    dc = jnp.sqrt(d2c_r[...])
    rb_c = jnp.exp(-((dc - cc[...]) ** 2) / wc[...])
    dn = jnp.sqrt(d2n_r[...])
    p2 = dn * dn
    p4 = p2 * p2
    u = 1.0 / p2
    v6 = 1.0 / (p4 * p2)
    rb2 = jnp.exp(-((u - c2[...]) ** 2) / w2[...])
    rb6 = jnp.exp(-((v6 - c6[...]) ** 2) / w6[...])
    ec[...] = (jnp.dot(xb[...], bondW[...], preferred_element_type=_f32, precision=jax.lax.Precision.HIGHEST)
               + jnp.dot(rb_c, rbfWc[...], preferred_element_type=_f32, precision=jax.lax.Precision.HIGHEST)
               + bconstc[...])
    en[...] = (jnp.dot(rb2, rbfW2[...], preferred_element_type=_f32, precision=jax.lax.Precision.HIGHEST)
               + jnp.dot(rb6, rbfW6[...], preferred_element_type=_f32, precision=jax.lax.Precision.HIGHEST)
               + bconstn[...])


def _upd_body(h_ref, s0c, s1c, s0n, s1n, dgc, dgn,
              w2c, b2c, w2n, b2n, wuc, buc, wun, bun,
              ho, cs):
    i = pl.program_id(0)
    hb = h_ref[...]
    Sc = s0c[...] + s1c[...]
    mc = jnp.dot(Sc, w2c[...], preferred_element_type=_f32, precision=jax.lax.Precision.HIGHEST) + dgc[...] * b2c[...]
    Sn = s0n[...] + s1n[...]
    mn = jnp.dot(Sn, w2n[...], preferred_element_type=_f32, precision=jax.lax.Precision.HIGHEST) + dgn[...] * b2n[...]
    hc = jnp.maximum(jnp.dot(hb + mc, wuc[...],
                             preferred_element_type=_f32, precision=jax.lax.Precision.HIGHEST) + buc[...], 0.0)
    hn = jnp.maximum(jnp.dot(hb + mn, wun[...],
                             preferred_element_type=_f32, precision=jax.lax.Precision.HIGHEST) + bun[...], 0.0)
    hnew = hc + hn
    ho[...] = hnew
    rows = i * RB + lax.broadcasted_iota(jnp.int32, (RB, 1), 0)
    cs[...] = jnp.sum(jnp.where(rows < N, hnew, 0.0),
                      axis=0, keepdims=True)[None]


def _head_body(parts, qkvW, qkvb, outW, outb, mW1, mb1, mw2, mb2, o_ref):
    r = jnp.sum(parts[...], axis=0)
    qkv = jnp.dot(r, qkvW[...], preferred_element_type=_f32, precision=jax.lax.Precision.HIGHEST) + qkvb[...]
    v = jnp.concatenate([qkv[:, 96 * t + 64:96 * t + 96] for t in range(4)],
                        axis=1)
    fin = jnp.dot(v, outW[...], preferred_element_type=_f32, precision=jax.lax.Precision.HIGHEST) + outb[...]
    o = jnp.maximum(jnp.dot(fin, mW1[...],
                            preferred_element_type=_f32, precision=jax.lax.Precision.HIGHEST) + mb1[...], 0.0)
    o_ref[...] = jnp.sum(o * mw2[...], axis=1, keepdims=True) + mb2[...]


def _row_spec(i):
    return (i, 0)


def _fix_spec(i):
    return (0, 0)


_h0_call = pl.pallas_call(
    _h0_body,
    grid=(GN,),
    in_specs=[pl.BlockSpec((RB, IN_DIM), _row_spec),
              pl.BlockSpec((IN_DIM, H), _fix_spec),
              pl.BlockSpec((1, H), _fix_spec)],
    out_specs=pl.BlockSpec((RB, H), _row_spec),
    out_shape=jax.ShapeDtypeStruct((NPAD, H), _f32),
)

_a_call = pl.pallas_call(
    _a_body,
    grid=(GN,),
    in_specs=[pl.BlockSpec((RB, H), _row_spec),
              pl.BlockSpec((4, H, H), lambda i: (0, 0, 0))],
    out_specs=[pl.BlockSpec((RB, H), _row_spec)] * 4,
    out_shape=[jax.ShapeDtypeStruct((NPAD, H), _f32)] * 4,
)

_e_call = pl.pallas_call(
    _e_body,
    grid=(GE,),
    in_specs=[pl.BlockSpec((RB, 1), _row_spec),
              pl.BlockSpec((RB, 10), _row_spec),
              pl.BlockSpec((RB, 1), _row_spec),
              pl.BlockSpec((1, NRBF), _fix_spec),
              pl.BlockSpec((1, NRBF), _fix_spec),
              pl.BlockSpec((1, NRBF), _fix_spec),
              pl.BlockSpec((1, 1), _fix_spec),
              pl.BlockSpec((1, 1), _fix_spec),
              pl.BlockSpec((1, 1), _fix_spec),
              pl.BlockSpec((NRBF, H), _fix_spec),
              pl.BlockSpec((10, H), _fix_spec),
              pl.BlockSpec((1, H), _fix_spec),
              pl.BlockSpec((NRBF, H), _fix_spec),
              pl.BlockSpec((NRBF, H), _fix_spec),
              pl.BlockSpec((1, H), _fix_spec)],
    out_specs=[pl.BlockSpec((RB, H), _row_spec)] * 2,
    out_shape=[jax.ShapeDtypeStruct((E, H), _f32)] * 2,
)

_upd_call = pl.pallas_call(
    _upd_body,
    grid=(GN,),
    in_specs=[pl.BlockSpec((RB, H), _row_spec),
              pl.BlockSpec((RB, H), _row_spec),
              pl.BlockSpec((RB, H), lambda i: (GN + i, 0)),
              pl.BlockSpec((RB, H), lambda i: (2 * GN + i, 0)),
              pl.BlockSpec((RB, H), lambda i: (3 * GN + i, 0)),
              pl.BlockSpec((RB, 1), _row_spec),
              pl.BlockSpec((RB, 1), _row_spec),
              pl.BlockSpec((H, H), _fix_spec),
              pl.BlockSpec((1, H), _fix_spec),
              pl.BlockSpec((H, H), _fix_spec),
              pl.BlockSpec((1, H), _fix_spec),
              pl.BlockSpec((H, H), _fix_spec),
              pl.BlockSpec((1, H), _fix_spec),
              pl.BlockSpec((H, H), _fix_spec),
              pl.BlockSpec((1, H), _fix_spec)],
    out_specs=[pl.BlockSpec((RB, H), _row_spec),
               pl.BlockSpec((1, 1, H), lambda i: (i, 0, 0))],
    out_shape=[jax.ShapeDtypeStruct((NPAD, H), _f32),
               jax.ShapeDtypeStruct((GN, 1, H), _f32)],
)

_head_call = pl.pallas_call(
    _head_body,
    out_shape=jax.ShapeDtypeStruct((1, 1), _f32),
)


def kernel(x, pos, edge_index_intra, x_bond, edge_index_inter, params):
    src_c, dst_c = edge_index_intra[0], edge_index_intra[1]
    src_n, dst_n = edge_index_inter[0], edge_index_inter[1]

    posp = jnp.zeros((NPAD, 3), _f32).at[:N].set(pos)
    xpad = jnp.zeros((NPAD, IN_DIM), _f32).at[:N].set(x)

    d2c, d2n, degc_p, degn_p = _sc_prep(
        posp[:, 0], posp[:, 1], posp[:, 2], src_c, dst_c, src_n, dst_n)

    deg_c = (degc_p[:NPAD, :1] + degc_p[NPAD:, :1])
    deg_n = (degn_p[:NPAD, :1] + degn_p[NPAD:, :1])

    lp = params['layers']
    bondW = jnp.stack([l['bond_W'] @ l['cov_W1'][256:384] for l in lp])
    bconstc = jnp.stack([(l['bond_b'] @ l['cov_W1'][256:384]
                          + l['cov_b1'])[None, :] for l in lp])
    rbfWc = jnp.stack([l['cov_W1'][384:448] for l in lp])
    rbfW2 = jnp.stack([l['ncov_W1'][256:320] for l in lp])
    rbfW6 = jnp.stack([l['ncov_W1'][320:384] for l in lp])
    bconstn = jnp.stack([l['ncov_b1'][None, :] for l in lp])

    cc = jnp.linspace(jnp.float32(1.0), jnp.float32(6.0), NRBF,
                      dtype=_f32)
    c2 = jnp.linspace(jnp.float32(1.0), jnp.float32(6.0) ** -2, NRBF,
                      dtype=_f32)
    c6 = jnp.linspace(jnp.float32(1.0), jnp.float32(6.0) ** -6, NRBF,
                      dtype=_f32)
    wc = ((cc[1] - cc[0]) ** 2).reshape(1, 1)
    w2 = ((c2[1] - c2[0]) ** 2).reshape(1, 1)
    w6 = ((c6[1] - c6[0]) ** 2).reshape(1, 1)

    d2c1, d2n1 = d2c.reshape(E, 1), d2n.reshape(E, 1)

    def e_layer(li):
        return _e_call(d2c1, x_bond, d2n1,
                       cc.reshape(1, NRBF), c2.reshape(1, NRBF),
                       c6.reshape(1, NRBF), wc, w2, w6,
                       rbfWc[li], bondW[li], bconstc[li],
                       rbfW2[li], rbfW6[li], bconstn[li])

    h = _h0_call(xpad, params['atom_W'], params['atom_b'][None, :])

    ecur = e_layer(0)
    cs = None
    for li, l in enumerate(lp):
        W4 = jnp.stack([l['cov_W1'][0:128], l['cov_W1'][128:256],
                        l['ncov_W1'][0:128], l['ncov_W1'][128:256]])
        a_ic, a_jc, a_in, a_jn = _a_call(h, W4)
        S = _sc_edge(src_c, dst_c, a_ic, a_jc, ecur[0],
                     src_n, dst_n, a_in, a_jn, ecur[1])
        if li + 1 < len(lp):
            ecur = e_layer(li + 1)
        h, cs = _upd_call(h, S, S, S, S, deg_c, deg_n,
                          l['cov_W2'], l['cov_b2'][None, :],
                          l['ncov_W2'], l['ncov_b2'][None, :],
                          l['upc_W'], l['upc_b'][None, :],
                          l['upn_W'], l['upn_b'][None, :])

    out = _head_call(cs, params['qkv_W'], params['qkv_b'][None, :],
                     params['out_W'], params['out_b'][None, :],
                     params['mlp_W1'], params['mlp_b1'][None, :],
                     params['mlp_W2'].reshape(1, H),
                     params['mlp_b2'].reshape(1, 1))
    return out.reshape(-1)


# speedup vs baseline: 3.5035x; 1.0456x over previous
import functools

import jax
import jax.numpy as jnp
from jax import lax
from jax.experimental import pallas as pl
from jax.experimental.pallas import tpu as pltpu
from jax.experimental.pallas import tpu_sc as plsc

N = 10000
NPAD = 10240
E = 320000
H = 128
IN_DIM = 35
NRBF = 64
NC, NS = 2, 16
NW = NC * NS
TE = E // NW
K = 80
NCHUNK = TE // K
ZR = 16
RB = 512
GN = NPAD // RB
GE = E // RB
P4 = 16

_f32 = jnp.float32

_mesh = plsc.VectorSubcoreMesh(core_axis_name="c", subcore_axis_name="s",
                               num_cores=NC, num_subcores=NS)


def _sc_prep_body(px_hbm, py_hbm, pz_hbm, srcc, dstc, srcn, dstn,
                  d2c_o, d2n_o, degc_o, degn_o,
                  sidx, didx, px_v, py_v, pz_v, d2_v, ones_v, zbuf,
                  degacc, sem):
    cid = lax.axis_index("c")
    sid = lax.axis_index("s")
    wid = cid * NS + sid
    zeros16 = jnp.zeros((16,), _f32)
    ones16 = jnp.ones((16,), _f32)

    pltpu.sync_copy(px_hbm, px_v)
    pltpu.sync_copy(py_hbm, py_v)
    pltpu.sync_copy(pz_hbm, pz_v)

    @pl.loop(0, ZR)
    def _init(r):
        for c in range(H // 16):
            zbuf[r, pl.ds(c * 16, 16)] = zeros16

    @pl.loop(0, K)
    def _init2(r):
        for c in range(H // 16):
            ones_v[r, pl.ds(c * 16, 16)] = ones16

    rpt = NPAD // NS

    for src, dst, d2_o, deg_o in ((srcc, dstc, d2c_o, degc_o),
                                  (srcn, dstn, d2n_o, degn_o)):
        @pl.loop(0, rpt // ZR)
        def _zacc(j):
            pltpu.sync_copy(zbuf, degacc.at[pl.ds(sid * rpt + j * ZR, ZR)])

        plsc.subcore_barrier()

        @pl.loop(0, NCHUNK)
        def _chunk(i):
            base = wid * TE + i * K
            pltpu.sync_copy(src.at[pl.ds(base, K)], sidx)
            pltpu.sync_copy(dst.at[pl.ds(base, K)], didx)
            for j in range(K // 16):
                iv_s = sidx[pl.ds(j * 16, 16)]
                iv_d = didx[pl.ds(j * 16, 16)]
                dx = (plsc.load_gather(px_v, [iv_s])
                      - plsc.load_gather(px_v, [iv_d]))
                dy = (plsc.load_gather(py_v, [iv_s])
                      - plsc.load_gather(py_v, [iv_d]))
                dz = (plsc.load_gather(pz_v, [iv_s])
                      - plsc.load_gather(pz_v, [iv_d]))
                d2_v[pl.ds(j * 16, 16)] = dx * dx + dy * dy + dz * dz
            pltpu.sync_copy(d2_v, d2_o.at[pl.ds(base, K)])
            pltpu.sync_copy(ones_v, degacc.at[didx], add=True)

        plsc.subcore_barrier()

        @pl.loop(0, rpt // ZR)
        def _dump(j):
            r0 = sid * rpt + j * ZR
            pltpu.sync_copy(degacc.at[pl.ds(r0, ZR)],
                            deg_o.at[pl.ds(cid * NPAD + r0, ZR)])

        plsc.subcore_barrier()


_sc_prep = functools.partial(
    pl.kernel,
    out_type=(jax.ShapeDtypeStruct((E,), _f32),
              jax.ShapeDtypeStruct((E,), _f32),
              jax.ShapeDtypeStruct((NC * NPAD, H), _f32),
              jax.ShapeDtypeStruct((NC * NPAD, H), _f32)),
    mesh=_mesh,
    scratch_types=[pltpu.VMEM((K,), jnp.int32),
                   pltpu.VMEM((K,), jnp.int32),
                   pltpu.VMEM((NPAD,), _f32),
                   pltpu.VMEM((NPAD,), _f32),
                   pltpu.VMEM((NPAD,), _f32),
                   pltpu.VMEM((K,), _f32),
                   pltpu.VMEM((K, H), _f32),
                   pltpu.VMEM((ZR, H), _f32),
                   pltpu.VMEM_SHARED((NPAD, H), _f32),
                   pltpu.SemaphoreType.DMA],
    compiler_params=pltpu.CompilerParams(needs_layout_passes=False),
)(_sc_prep_body)


K2 = 40
NCHUNK2 = TE // K2


def _sc_edge_body(srcc, dstc, aic_t, ajc_t, ec_t,
                  srcn, dstn, ain_t, ajn_t, en_t, out_o,
                  sA, dA, sB, dB, ai0, aj0, e0, ai1, aj1, e1,
                  acc, semIA, semIB, semA, semB):
    cid = lax.axis_index("c")
    sid = lax.axis_index("s")
    wid = cid * NS + sid
    zeros16 = jnp.zeros((16,), _f32)
    rpt = NPAD // NS
    tb = wid * TE

    for bi, (src, dst, ai_t, aj_t, e_t, ob) in enumerate(
            ((srcc, dstc, aic_t, ajc_t, ec_t, 0),
             (srcn, dstn, ain_t, ajn_t, en_t, NC * NPAD))):
        @pl.loop(0, K2)
        def _z(r):
            for c in range(H // 16):
                e0[r, pl.ds(c * 16, 16)] = zeros16

        @pl.loop(0, rpt // K2)
        def _zacc(j):
            pltpu.sync_copy(e0, acc.at[pl.ds(sid * rpt + j * K2, K2)])

        plsc.subcore_barrier()

        def issue_idx(c, sb, db, sem):
            cc = jnp.minimum(c, NCHUNK2 - 1)
            pltpu.async_copy(src.at[pl.ds(tb + cc * K2, K2)], sb, sem)
            pltpu.async_copy(dst.at[pl.ds(tb + cc * K2, K2)], db, sem)

        def drain_idx(sb, db, sem):
            pltpu.make_async_copy(src.at[pl.ds(0, K2)], sb, sem).wait()
            pltpu.make_async_copy(dst.at[pl.ds(0, K2)], db, sem).wait()

        def issue_g(c, sb, db, ai_b, aj_b, e_b, sem):
            cc = jnp.minimum(c, NCHUNK2 - 1)
            pltpu.async_copy(ai_t.at[db], ai_b, sem)
            pltpu.async_copy(aj_t.at[sb], aj_b, sem)
            pltpu.async_copy(e_t.at[pl.ds(tb + cc * K2, K2)], e_b, sem)

        def drain_g(ai_b, aj_b, e_b, sem):
            pltpu.make_async_copy(ai_t.at[pl.ds(0, K2)], ai_b, sem).wait()
            pltpu.make_async_copy(aj_t.at[pl.ds(0, K2)], aj_b, sem).wait()
            pltpu.make_async_copy(e_t.at[pl.ds(0, K2)], e_b, sem).wait()

        def compute_scatter(db, ai_b, aj_b, e_b):
            @pl.loop(0, K2 // 2)
            def _row(r):
                r0 = 2 * r
                for rr in (r0, r0 + 1):
                    for cc in range(H // 16):
                        sl = pl.ds(cc * 16, 16)
                        s = ai_b[rr, sl] + aj_b[rr, sl] + e_b[rr, sl]
                        ai_b[rr, sl] = jnp.maximum(s, 0.0)

            pltpu.sync_copy(ai_b, acc.at[db], add=True)

        issue_idx(0, sA, dA, semIA)
        issue_idx(1, sB, dB, semIB)
        drain_idx(sA, dA, semIA)
        issue_g(0, sA, dA, ai0, aj0, e0, semA)

        @pl.loop(0, NCHUNK2 // 2)
        def _pair(k):
            c0 = 2 * k
            drain_g(ai0, aj0, e0, semA)
            drain_idx(sB, dB, semIB)
            issue_g(c0 + 1, sB, dB, ai1, aj1, e1, semB)
            compute_scatter(dA, ai0, aj0, e0)
            issue_idx(c0 + 2, sA, dA, semIA)
            drain_g(ai1, aj1, e1, semB)
            drain_idx(sA, dA, semIA)
            issue_g(c0 + 2, sA, dA, ai0, aj0, e0, semA)
            compute_scatter(dB, ai1, aj1, e1)
            issue_idx(c0 + 3, sB, dB, semIB)

        drain_g(ai0, aj0, e0, semA)
        drain_idx(sB, dB, semIB)

        plsc.subcore_barrier()

        pltpu.sync_copy(acc.at[pl.ds(sid * rpt, rpt)],
                        out_o.at[pl.ds(ob + cid * NPAD + sid * rpt, rpt)])

        plsc.subcore_barrier()


_sc_edge = functools.partial(
    pl.kernel,
    out_type=jax.ShapeDtypeStruct((2 * NC * NPAD, H), _f32),
    mesh=_mesh,
    scratch_types=[pltpu.VMEM((K2,), jnp.int32),
                   pltpu.VMEM((K2,), jnp.int32),
                   pltpu.VMEM((K2,), jnp.int32),
                   pltpu.VMEM((K2,), jnp.int32),
                   pltpu.VMEM((K2, H), _f32),
                   pltpu.VMEM((K2, H), _f32),
                   pltpu.VMEM((K2, H), _f32),
                   pltpu.VMEM((K2, H), _f32),
                   pltpu.VMEM((K2, H), _f32),
                   pltpu.VMEM((K2, H), _f32),
                   pltpu.VMEM_SHARED((NPAD, H), _f32),
                   pltpu.SemaphoreType.DMA,
                   pltpu.SemaphoreType.DMA,
                   pltpu.SemaphoreType.DMA,
                   pltpu.SemaphoreType.DMA],
    compiler_params=pltpu.CompilerParams(needs_layout_passes=False),
)(_sc_edge_body)


_bf16 = jnp.bfloat16


def _bdot(a, b):
    return jnp.dot(a.astype(_bf16), b.astype(_bf16),
                   preferred_element_type=_f32)


def _wdot(a, b):
    return jnp.dot(a, b.astype(_bf16).astype(_f32),
                   preferred_element_type=_f32,
                   precision=jax.lax.Precision.HIGHEST)


def _h0_body(x_ref, w_ref, b_ref, o_ref):
    o_ref[...] = _bdot(x_ref[...], w_ref[...]) + b_ref[...]


def _a_body(h_ref, w_ref, o0, o1, o2, o3):
    hb = h_ref[...]
    for idx, o in enumerate((o0, o1, o2, o3)):
        o[...] = _wdot(hb, w_ref[idx])


def _e_body(d2c_r, xb, d2n_r,
            cc, c2, c6, wc, w2, w6,
            rbfWc, bondW, bondb, w1cb, bconstc, rbfW2, rbfW6, bconstn,
            ec, en):
    dc = jnp.sqrt(d2c_r[...])
    rb_c = jnp.exp(-((dc - cc[...]) ** 2) / wc[...])
    dn = jnp.sqrt(d2n_r[...])
    p2 = dn * dn
    p4 = p2 * p2
    u = 1.0 / p2
    v6 = 1.0 / (p4 * p2)
    rb2 = jnp.exp(-((u - c2[...]) ** 2) / w2[...])
    rb6 = jnp.exp(-((v6 - c6[...]) ** 2) / w6[...])
    bond = _bdot(xb[...], bondW[...]) + bondb[...]
    ec[...] = (_bdot(bond, w1cb[...])
               + _wdot(rb_c, rbfWc[...])
               + bconstc[...])
    en[...] = (_wdot(rb2, rbfW2[...])
               + _wdot(rb6, rbfW6[...])
               + bconstn[...])


def _upd_body(h_ref, s0c, s1c, s0n, s1n, dgc, dgn,
              w2c, b2c, w2n, b2n, wuc, buc, wun, bun,
              ho, cs):
    i = pl.program_id(0)
    hb = h_ref[...]
    Sc = s0c[...] + s1c[...]
    mc = _wdot(Sc, w2c[...]) + dgc[...] * b2c[...]
    Sn = s0n[...] + s1n[...]
    mn = _wdot(Sn, w2n[...]) + dgn[...] * b2n[...]
    hc = jnp.maximum(_wdot(hb + mc, wuc[...]) + buc[...], 0.0)
    hn = jnp.maximum(_wdot(hb + mn, wun[...]) + bun[...], 0.0)
    hnew = hc + hn
    ho[...] = hnew
    rows = i * RB + lax.broadcasted_iota(jnp.int32, (RB, 1), 0)
    cs[...] = jnp.sum(jnp.where(rows < N, hnew, 0.0),
                      axis=0, keepdims=True)[None]


def _head_body(parts, qkvW, qkvb, outW, outb, mW1, mb1, mw2, mb2, o_ref):
    r = jnp.sum(parts[...], axis=0)
    qkv = _bdot(r, qkvW[...]) + qkvb[...]
    v = jnp.concatenate([qkv[:, 96 * t + 64:96 * t + 96] for t in range(4)],
                        axis=1)
    v = v.astype(_bf16).astype(_f32)
    fin = _bdot(v, outW[...]) + outb[...]
    o = jnp.maximum(_bdot(fin, mW1[...]) + mb1[...], 0.0)
    o_ref[...] = jnp.sum((o.astype(_bf16).astype(_f32))
                         * (mw2[...].astype(_bf16).astype(_f32)),
                         axis=1, keepdims=True) + mb2[...]


def _row_spec(i):
    return (i, 0)


def _fix_spec(i):
    return (0, 0)


_h0_call = pl.pallas_call(
    _h0_body,
    grid=(GN,),
    in_specs=[pl.BlockSpec((RB, IN_DIM), _row_spec),
              pl.BlockSpec((IN_DIM, H), _fix_spec),
              pl.BlockSpec((1, H), _fix_spec)],
    out_specs=pl.BlockSpec((RB, H), _row_spec),
    out_shape=jax.ShapeDtypeStruct((NPAD, H), _f32),
)

_a_call = pl.pallas_call(
    _a_body,
    grid=(GN,),
    in_specs=[pl.BlockSpec((RB, H), _row_spec),
              pl.BlockSpec((4, H, H), lambda i: (0, 0, 0))],
    out_specs=[pl.BlockSpec((RB, H), _row_spec)] * 4,
    out_shape=[jax.ShapeDtypeStruct((NPAD, H), _f32)] * 4,
)

_e_call = pl.pallas_call(
    _e_body,
    grid=(GE,),
    in_specs=[pl.BlockSpec((RB, 1), _row_spec),
              pl.BlockSpec((RB, 10), _row_spec),
              pl.BlockSpec((RB, 1), _row_spec),
              pl.BlockSpec((1, NRBF), _fix_spec),
              pl.BlockSpec((1, NRBF), _fix_spec),
              pl.BlockSpec((1, NRBF), _fix_spec),
              pl.BlockSpec((1, 1), _fix_spec),
              pl.BlockSpec((1, 1), _fix_spec),
              pl.BlockSpec((1, 1), _fix_spec),
              pl.BlockSpec((NRBF, H), _fix_spec),
              pl.BlockSpec((10, H), _fix_spec),
              pl.BlockSpec((1, H), _fix_spec),
              pl.BlockSpec((H, H), _fix_spec),
              pl.BlockSpec((1, H), _fix_spec),
              pl.BlockSpec((NRBF, H), _fix_spec),
              pl.BlockSpec((NRBF, H), _fix_spec),
              pl.BlockSpec((1, H), _fix_spec)],
    out_specs=[pl.BlockSpec((RB, H), _row_spec)] * 2,
    out_shape=[jax.ShapeDtypeStruct((E, H), _f32)] * 2,
)

_upd_call = pl.pallas_call(
    _upd_body,
    grid=(GN,),
    in_specs=[pl.BlockSpec((RB, H), _row_spec),
              pl.BlockSpec((RB, H), _row_spec),
              pl.BlockSpec((RB, H), lambda i: (GN + i, 0)),
              pl.BlockSpec((RB, H), lambda i: (2 * GN + i, 0)),
              pl.BlockSpec((RB, H), lambda i: (3 * GN + i, 0)),
              pl.BlockSpec((RB, 1), _row_spec),
              pl.BlockSpec((RB, 1), _row_spec),
              pl.BlockSpec((H, H), _fix_spec),
              pl.BlockSpec((1, H), _fix_spec),
              pl.BlockSpec((H, H), _fix_spec),
              pl.BlockSpec((1, H), _fix_spec),
              pl.BlockSpec((H, H), _fix_spec),
              pl.BlockSpec((1, H), _fix_spec),
              pl.BlockSpec((H, H), _fix_spec),
              pl.BlockSpec((1, H), _fix_spec)],
    out_specs=[pl.BlockSpec((RB, H), _row_spec),
               pl.BlockSpec((1, 1, H), lambda i: (i, 0, 0))],
    out_shape=[jax.ShapeDtypeStruct((NPAD, H), _f32),
               jax.ShapeDtypeStruct((GN, 1, H), _f32)],
)

_head_call = pl.pallas_call(
    _head_body,
    out_shape=jax.ShapeDtypeStruct((1, 1), _f32),
)


def kernel(x, pos, edge_index_intra, x_bond, edge_index_inter, params):
    src_c, dst_c = edge_index_intra[0], edge_index_intra[1]
    src_n, dst_n = edge_index_inter[0], edge_index_inter[1]

    posp = jnp.zeros((NPAD, 3), _f32).at[:N].set(pos)
    xpad = jnp.zeros((NPAD, IN_DIM), _f32).at[:N].set(x)

    d2c, d2n, degc_p, degn_p = _sc_prep(
        posp[:, 0], posp[:, 1], posp[:, 2], src_c, dst_c, src_n, dst_n)

    deg_c = (degc_p[:NPAD, :1] + degc_p[NPAD:, :1])
    deg_n = (degn_p[:NPAD, :1] + degn_p[NPAD:, :1])

    lp = params['layers']
    bondW = jnp.stack([l['bond_W'] for l in lp])
    bondb = jnp.stack([l['bond_b'][None, :] for l in lp])
    w1cb = jnp.stack([l['cov_W1'][256:384] for l in lp])
    bconstc = jnp.stack([l['cov_b1'][None, :] for l in lp])
    rbfWc = jnp.stack([l['cov_W1'][384:448] for l in lp])
    rbfW2 = jnp.stack([l['ncov_W1'][256:320] for l in lp])
    rbfW6 = jnp.stack([l['ncov_W1'][320:384] for l in lp])
    bconstn = jnp.stack([l['ncov_b1'][None, :] for l in lp])

    cc = jnp.linspace(jnp.float32(1.0), jnp.float32(6.0), NRBF,
                      dtype=_f32)
    c2 = jnp.linspace(jnp.float32(1.0), jnp.float32(6.0) ** -2, NRBF,
                      dtype=_f32)
    c6 = jnp.linspace(jnp.float32(1.0), jnp.float32(6.0) ** -6, NRBF,
                      dtype=_f32)
    wc = ((cc[1] - cc[0]) ** 2).reshape(1, 1)
    w2 = ((c2[1] - c2[0]) ** 2).reshape(1, 1)
    w6 = ((c6[1] - c6[0]) ** 2).reshape(1, 1)

    d2c1, d2n1 = d2c.reshape(E, 1), d2n.reshape(E, 1)

    def e_layer(li):
        return _e_call(d2c1, x_bond, d2n1,
                       cc.reshape(1, NRBF), c2.reshape(1, NRBF),
                       c6.reshape(1, NRBF), wc, w2, w6,
                       rbfWc[li], bondW[li], bondb[li], w1cb[li],
                       bconstc[li], rbfW2[li], rbfW6[li], bconstn[li])

    h = _h0_call(xpad, params['atom_W'], params['atom_b'][None, :])

    ecur = e_layer(0)
    cs = None
    for li, l in enumerate(lp):
        W4 = jnp.stack([l['cov_W1'][0:128], l['cov_W1'][128:256],
                        l['ncov_W1'][0:128], l['ncov_W1'][128:256]])
        a_ic, a_jc, a_in, a_jn = _a_call(h, W4)
        S = _sc_edge(src_c, dst_c, a_ic, a_jc, ecur[0],
                     src_n, dst_n, a_in, a_jn, ecur[1])
        if li + 1 < len(lp):
            ecur = e_layer(li + 1)
        h, cs = _upd_call(h, S, S, S, S, deg_c, deg_n,
                          l['cov_W2'], l['cov_b2'][None, :],
                          l['ncov_W2'], l['ncov_b2'][None, :],
                          l['upc_W'], l['upc_b'][None, :],
                          l['upn_W'], l['upn_b'][None, :])

    out = _head_call(cs, params['qkv_W'], params['qkv_b'][None, :],
                     params['out_W'], params['out_b'][None, :],
                     params['mlp_W1'], params['mlp_b1'][None, :],
                     params['mlp_W2'].reshape(1, H),
                     params['mlp_b2'].reshape(1, 1))
    return out.reshape(-1)
